# baseline (speedup 1.0000x reference)
import functools

import jax
import jax.numpy as jnp
from jax import lax
from jax.experimental import pallas as pl
from jax.experimental.pallas import tpu as pltpu
from jax.experimental.pallas import tpu_sc as plsc

NUM_ROWS = 129
DIM = 128
B = 4096
U = 200
TOTAL = B * U

NC = 2
NS = 16
NW = NC * NS
PER_W = TOTAL // NW
G = 128
KPC = 1
CHUNK = G * KPC
NBUF = 4
NCHUNK = PER_W // CHUNK
NOUTER = NCHUNK // NBUF
IDX_ROWS_PER_W = PER_W // G


def _sc_gather(idx2d, table):
    mesh = plsc.VectorSubcoreMesh(core_axis_name="c", subcore_axis_name="s")

    @functools.partial(
        pl.kernel,
        mesh=mesh,
        out_type=jax.ShapeDtypeStruct((TOTAL, DIM), jnp.float32),
        scratch_types=[
            pltpu.VMEM((IDX_ROWS_PER_W, G), jnp.int32),
            pltpu.VMEM((CHUNK, DIM), jnp.float32),
            pltpu.VMEM((CHUNK, DIM), jnp.float32),
            pltpu.VMEM((CHUNK, DIM), jnp.float32),
            pltpu.VMEM((CHUNK, DIM), jnp.float32),
            pltpu.VMEM_SHARED((NUM_ROWS, DIM), jnp.float32),
            pltpu.SemaphoreType.DMA,
            pltpu.SemaphoreType.DMA,
            pltpu.SemaphoreType.DMA,
            pltpu.SemaphoreType.DMA,
            pltpu.SemaphoreType.DMA,
        ],
    )
    def k(idx_hbm, table_hbm, out_hbm, idx_v, rows_v0, rows_v1, rows_v2,
          rows_v3, table_sh, gsem, ssem0, ssem1, ssem2, ssem3):
        sid = lax.axis_index("s")
        wid = sid * NC + lax.axis_index("c")
        row0 = wid * IDX_ROWS_PER_W
        out0 = wid * PER_W
        rows_bufs = (rows_v0, rows_v1, rows_v2, rows_v3)
        ssems = (ssem0, ssem1, ssem2, ssem3)

        @pl.when(sid == 0)
        def _stage_table():
            pltpu.sync_copy(table_hbm, table_sh)

        pltpu.sync_copy(idx_hbm.at[pl.ds(row0, IDX_ROWS_PER_W)], idx_v)
        plsc.subcore_barrier()

        def body(j, carry):
            for b in range(NBUF):
                rows_v = rows_bufs[b]
                ssem = ssems[b]
                c = j * NBUF + b

                @pl.when(j > 0)
                def _drain():
                    pltpu.make_async_copy(
                        rows_v, out_hbm.at[pl.ds(out0, CHUNK)], ssem
                    ).wait()

                cps = [
                    pltpu.async_copy(
                        table_sh.at[idx_v.at[c * KPC + t]],
                        rows_v.at[pl.ds(t * G, G)],
                        gsem,
                    )
                    for t in range(KPC)
                ]
                for cp in cps:
                    cp.wait()

                pltpu.async_copy(
                    rows_v, out_hbm.at[pl.ds(out0 + c * CHUNK, CHUNK)], ssem
                )
            return carry

        lax.fori_loop(0, NOUTER, body, 0)

        for b in range(NBUF):
            pltpu.make_async_copy(
                rows_bufs[b], out_hbm.at[pl.ds(out0, CHUNK)], ssems[b]
            ).wait()

    return k(idx2d, table)


def kernel(distance_bin_ids, embedding):
    idx2d = distance_bin_ids.reshape(TOTAL // G, G).astype(jnp.int32)
    out = _sc_gather(idx2d, embedding)
    return out.reshape(B, U, DIM)


# speedup vs baseline: 1.0393x; 1.0393x over previous
import functools

import jax
import jax.numpy as jnp
from jax import lax
from jax.experimental import pallas as pl
from jax.experimental.pallas import tpu as pltpu
from jax.experimental.pallas import tpu_sc as plsc

NUM_ROWS = 129
DIM = 128
B = 4096
U = 200
TOTAL = B * U

NC = 2
NS = 16
NW = NC * NS
PER_W = TOTAL // NW
G = 128
KPC = 2
CHUNK = G * KPC
NBUF = 3
NCHUNK = PER_W // CHUNK
NOUTER = NCHUNK // NBUF
NPEEL = NCHUNK - NOUTER * NBUF
IDX_ROWS_PER_W = PER_W // G


def _sc_gather(idx2d, table):
    mesh = plsc.VectorSubcoreMesh(core_axis_name="c", subcore_axis_name="s")

    @functools.partial(
        pl.kernel,
        mesh=mesh,
        out_type=jax.ShapeDtypeStruct((TOTAL, DIM), jnp.float32),
        scratch_types=[
            pltpu.VMEM((IDX_ROWS_PER_W, G), jnp.int32),
            pltpu.VMEM((CHUNK, DIM), jnp.float32),
            pltpu.VMEM((CHUNK, DIM), jnp.float32),
            pltpu.VMEM((CHUNK, DIM), jnp.float32),
            pltpu.VMEM_SHARED((NUM_ROWS, DIM), jnp.float32),
            pltpu.SemaphoreType.DMA,
            pltpu.SemaphoreType.DMA,
            pltpu.SemaphoreType.DMA,
            pltpu.SemaphoreType.DMA,
        ],
    )
    def k(idx_hbm, table_hbm, out_hbm, idx_v, rows_v0, rows_v1, rows_v2,
          table_sh, gsem, ssem0, ssem1, ssem2):
        sid = lax.axis_index("s")
        wid = sid * NC + lax.axis_index("c")
        row0 = wid * IDX_ROWS_PER_W
        out0 = wid * PER_W
        rows_bufs = (rows_v0, rows_v1, rows_v2)
        ssems = (ssem0, ssem1, ssem2)

        @pl.when(sid == 0)
        def _stage_table():
            pltpu.sync_copy(table_hbm, table_sh)

        pltpu.sync_copy(idx_hbm.at[pl.ds(row0, IDX_ROWS_PER_W)], idx_v)
        plsc.subcore_barrier()

        def do_chunk(c, rows_v, ssem, first):
            if not first:
                pltpu.make_async_copy(
                    rows_v, out_hbm.at[pl.ds(out0, CHUNK)], ssem
                ).wait()
            cps = [
                pltpu.async_copy(
                    table_sh.at[idx_v.at[c * KPC + t]],
                    rows_v.at[pl.ds(t * G, G)],
                    gsem,
                )
                for t in range(KPC)
            ]
            for cp in cps:
                cp.wait()
            pltpu.async_copy(
                rows_v, out_hbm.at[pl.ds(out0 + c * CHUNK, CHUNK)], ssem
            )

        def body(j, carry):
            for b in range(NBUF):
                rows_v = rows_bufs[b]
                ssem = ssems[b]
                c = j * NBUF + b

                @pl.when(j > 0)
                def _steady():
                    do_chunk(c, rows_v, ssem, first=False)

                @pl.when(j == 0)
                def _first():
                    do_chunk(c, rows_v, ssem, first=True)

            return carry

        lax.fori_loop(0, NOUTER, body, 0)

        for p in range(NPEEL):
            do_chunk(NOUTER * NBUF + p, rows_bufs[p], ssems[p], first=False)

        for b in range(NBUF):
            pltpu.make_async_copy(
                rows_bufs[b], out_hbm.at[pl.ds(out0, CHUNK)], ssems[b]
            ).wait()

    return k(idx2d, table)


def kernel(distance_bin_ids, embedding):
    idx2d = distance_bin_ids.reshape(TOTAL // G, G).astype(jnp.int32)
    out = _sc_gather(idx2d, embedding)
    return out.reshape(B, U, DIM)


# speedup vs baseline: 1.0428x; 1.0034x over previous
import functools

import jax
import jax.numpy as jnp
from jax import lax
from jax.experimental import pallas as pl
from jax.experimental.pallas import tpu as pltpu
from jax.experimental.pallas import tpu_sc as plsc

NUM_ROWS = 129
DIM = 128
B = 4096
U = 200
TOTAL = B * U

NC = 2
NS = 16
NW = NC * NS
PER_W = TOTAL // NW
G = 128
KPC = 2
CHUNK = G * KPC
NBUF = 2
NCHUNK = PER_W // CHUNK
NOUTER = NCHUNK // NBUF
IDX_ROWS_PER_W = PER_W // G


def _sc_gather(idx2d, table):
    mesh = plsc.VectorSubcoreMesh(core_axis_name="c", subcore_axis_name="s")

    @functools.partial(
        pl.kernel,
        mesh=mesh,
        out_type=jax.ShapeDtypeStruct((TOTAL, DIM), jnp.float32),
        scratch_types=[
            pltpu.VMEM((IDX_ROWS_PER_W, G), jnp.int32),
            pltpu.VMEM((CHUNK, DIM), jnp.float32),
            pltpu.VMEM((CHUNK, DIM), jnp.float32),
            pltpu.VMEM_SHARED((NUM_ROWS, DIM), jnp.float32),
            pltpu.SemaphoreType.DMA,
            pltpu.SemaphoreType.DMA,
            pltpu.SemaphoreType.DMA,
        ],
    )
    def k(idx_hbm, table_hbm, out_hbm, idx_v, rows_v0, rows_v1, table_sh,
          gsem, ssem0, ssem1):
        sid = lax.axis_index("s")
        wid = sid * NC + lax.axis_index("c")
        row0 = wid * IDX_ROWS_PER_W
        out0 = wid * PER_W
        rows_bufs = (rows_v0, rows_v1)
        ssems = (ssem0, ssem1)

        @pl.when(sid == 0)
        def _stage_table():
            pltpu.sync_copy(table_hbm, table_sh)

        pltpu.sync_copy(idx_hbm.at[pl.ds(row0, IDX_ROWS_PER_W)], idx_v)
        plsc.subcore_barrier()

        def body(j, carry):
            for b in range(NBUF):
                rows_v = rows_bufs[b]
                ssem = ssems[b]
                c = j * NBUF + b

                @pl.when(j > 0)
                def _drain():
                    pltpu.make_async_copy(
                        rows_v, out_hbm.at[pl.ds(out0, CHUNK)], ssem
                    ).wait()

                cps = [
                    pltpu.async_copy(
                        table_sh.at[idx_v.at[c * KPC + t]],
                        rows_v.at[pl.ds(t * G, G)],
                        gsem,
                    )
                    for t in range(KPC)
                ]
                for cp in cps:
                    cp.wait()

                pltpu.async_copy(
                    rows_v, out_hbm.at[pl.ds(out0 + c * CHUNK, CHUNK)], ssem
                )
            return carry

        lax.fori_loop(0, NOUTER, body, 0)

        for b in range(NBUF):
            pltpu.make_async_copy(
                rows_bufs[b], out_hbm.at[pl.ds(out0, CHUNK)], ssems[b]
            ).wait()

    return k(idx2d, table)


def kernel(distance_bin_ids, embedding):
    idx2d = distance_bin_ids.reshape(TOTAL // G, G).astype(jnp.int32)
    out = _sc_gather(idx2d, embedding)
    return out.reshape(B, U, DIM)
